# Initial kernel scaffold; baseline (speedup 1.0000x reference)
import jax
import jax.numpy as jnp
from jax.experimental import pallas as pl


def kernel(input, labels, emb_weight, lin_weight, lin_bias):
    raise NotImplementedError("write your pallas kernel here")


# speedup vs baseline: 156.1177x; 156.1177x over previous
import functools

import jax
import jax.numpy as jnp
from jax import lax
from jax.experimental import pallas as pl
from jax.experimental.pallas import tpu as pltpu
from jax.experimental.pallas import tpu_sc as plsc

NC = 2
NS = 16
L = 16
NW = NC * NS

B = 16384
H = 200
VOCAB = 500
VPAD = 512
EMB = 10
EPAD = 16

RPW = B // NW
TPW = RPW * H
NVEC = TPW // L


def _prep_body(emb_ref, w_ref, bias_ref, out_ref):
    e = emb_ref[...]
    n2 = jnp.sum(e * e, axis=1, keepdims=True)
    norm = jnp.sqrt(n2)
    scale = jnp.minimum(1.0, 1.0 / jnp.maximum(norm, 1e-12))
    te = e * scale
    p = lax.dot_general(w_ref[...], te, (((1,), (1,)), ((), ())),
                        preferred_element_type=jnp.float32)
    out_ref[...] = p + bias_ref[...]


_prep = pl.pallas_call(
    _prep_body,
    out_shape=jax.ShapeDtypeStruct((2, VPAD), jnp.float32),
)


def _sc_body(tok_hbm, lab_hbm, p_hbm, out_hbm, tok_v, lab_v, p_v, acc_v):
    wid = lax.axis_index("s") * NC + lax.axis_index("c")
    pltpu.sync_copy(p_hbm, p_v)
    pltpu.sync_copy(lab_hbm.at[pl.ds(wid * RPW, RPW)], lab_v)
    pltpu.sync_copy(tok_hbm.at[pl.ds(wid * TPW, TPW)], tok_v)

    def body(i, acc):
        off = i * L
        tok = tok_v[pl.ds(off, L)]
        row = (lax.iota(jnp.int32, L) + off) // H
        labv = plsc.load_gather(lab_v, [row])
        fidx = tok + labv * VPAD
        val = plsc.load_gather(p_v, [fidx])
        return acc + val

    acc = lax.fori_loop(0, NVEC, body, jnp.zeros((L,), jnp.float32))
    acc_v[...] = acc
    pltpu.sync_copy(acc_v, out_hbm.at[pl.ds(wid * L, L)])


_gather_sum = functools.partial(
    pl.kernel,
    out_type=jax.ShapeDtypeStruct((NW * L,), jnp.float32),
    mesh=plsc.VectorSubcoreMesh(core_axis_name="c", subcore_axis_name="s"),
    compiler_params=pltpu.CompilerParams(needs_layout_passes=False),
    scratch_types=[
        pltpu.VMEM((TPW,), jnp.int32),
        pltpu.VMEM((RPW,), jnp.int32),
        pltpu.VMEM((2 * VPAD,), jnp.float32),
        pltpu.VMEM((L,), jnp.float32),
    ],
)(_sc_body)


def kernel(input, labels, emb_weight, lin_weight, lin_bias):
    tok = jnp.reshape(input.astype(jnp.int32), (B * H,))
    lab = labels.astype(jnp.int32)

    emb_pad = jnp.zeros((VPAD, EPAD), jnp.float32).at[:VOCAB, :EMB].set(
        emb_weight.astype(jnp.float32))
    w_pad = jnp.zeros((2, EPAD), jnp.float32).at[:, :EMB].set(
        lin_weight.astype(jnp.float32))
    bias2d = jnp.broadcast_to(
        (lin_bias.astype(jnp.float32) / H)[:, None], (2, VPAD))

    p_flat = jnp.reshape(_prep(emb_pad, w_pad, bias2d), (2 * VPAD,))
    partials = _gather_sum(tok, lab, p_flat)
    return -jnp.sum(partials) / B


# speedup vs baseline: 185.7216x; 1.1896x over previous
import functools

import jax
import jax.numpy as jnp
from jax import lax
from jax.experimental import pallas as pl
from jax.experimental.pallas import tpu as pltpu
from jax.experimental.pallas import tpu_sc as plsc

NC = 2
NS = 16
L = 16
NW = NC * NS

B = 16384
H = 200
VOCAB = 500
VPAD = 512
EMB = 10
EPAD = 16

RPW = B // NW
TPW = RPW * H
NVEC = TPW // L


def _prep_body(emb_ref, w_ref, bias_ref, out_ref):
    e = emb_ref[...]
    n2 = jnp.sum(e * e, axis=1, keepdims=True)
    norm = jnp.sqrt(n2)
    scale = jnp.minimum(1.0, 1.0 / jnp.maximum(norm, 1e-12))
    te = e * scale
    p = lax.dot_general(w_ref[...], te, (((1,), (1,)), ((), ())),
                        preferred_element_type=jnp.float32)
    out_ref[...] = p + bias_ref[...]


_prep = pl.pallas_call(
    _prep_body,
    out_shape=jax.ShapeDtypeStruct((2, VPAD), jnp.float32),
)


NPAIR = RPW // 2


def _sc_body(tok_hbm, lab_hbm, p_hbm, out_hbm, tok_v, lab_v, p_v, acc_v):
    wid = lax.axis_index("s") * NC + lax.axis_index("c")
    pltpu.sync_copy(p_hbm, p_v)
    pltpu.sync_copy(lab_hbm.at[pl.ds(wid * RPW, RPW)], lab_v)
    pltpu.sync_copy(tok_hbm.at[pl.ds(wid * TPW, TPW)], tok_v)

    zero16 = jnp.zeros((L,), jnp.int32)
    mask12 = lax.iota(jnp.int32, L) >= 8

    def body(p, accs):
        base = p * (2 * H)
        lab0 = plsc.load_gather(lab_v, [zero16 + 2 * p])
        lab1 = plsc.load_gather(lab_v, [zero16 + (2 * p + 1)])
        off0 = lab0 * VPAD
        off1 = lab1 * VPAD
        offm = jnp.where(mask12, off1, off0)
        a = list(accs)
        for j in range(25):
            tok = tok_v[pl.ds(base + j * L, L)]
            offv = off0 if j < 12 else (offm if j == 12 else off1)
            a[j % 4] = a[j % 4] + plsc.load_gather(p_v, [tok + offv])
        return tuple(a)

    z = jnp.zeros((L,), jnp.float32)
    a0, a1, a2, a3 = lax.fori_loop(0, NPAIR, body, (z, z, z, z))
    acc_v[...] = (a0 + a1) + (a2 + a3)
    pltpu.sync_copy(acc_v, out_hbm.at[pl.ds(wid * L, L)])


_gather_sum = functools.partial(
    pl.kernel,
    out_type=jax.ShapeDtypeStruct((NW * L,), jnp.float32),
    mesh=plsc.VectorSubcoreMesh(core_axis_name="c", subcore_axis_name="s"),
    compiler_params=pltpu.CompilerParams(needs_layout_passes=False),
    scratch_types=[
        pltpu.VMEM((TPW,), jnp.int32),
        pltpu.VMEM((RPW,), jnp.int32),
        pltpu.VMEM((2 * VPAD,), jnp.float32),
        pltpu.VMEM((L,), jnp.float32),
    ],
)(_sc_body)


def kernel(input, labels, emb_weight, lin_weight, lin_bias):
    tok = jnp.reshape(input.astype(jnp.int32), (B * H,))
    lab = labels.astype(jnp.int32)

    emb_pad = jnp.zeros((VPAD, EPAD), jnp.float32).at[:VOCAB, :EMB].set(
        emb_weight.astype(jnp.float32))
    w_pad = jnp.zeros((2, EPAD), jnp.float32).at[:, :EMB].set(
        lin_weight.astype(jnp.float32))
    bias2d = jnp.broadcast_to(
        (lin_bias.astype(jnp.float32) / H)[:, None], (2, VPAD))

    p_flat = jnp.reshape(_prep(emb_pad, w_pad, bias2d), (2 * VPAD,))
    partials = _gather_sum(tok, lab, p_flat)
    return -jnp.sum(partials) / B
